# baseline (speedup 1.0000x reference)
import functools

import jax
import jax.numpy as jnp
from jax import lax
from jax.experimental import pallas as pl
from jax.experimental.pallas import tpu as pltpu
from jax.experimental.pallas import tpu_sc as plsc

_NUM_EMBEDDINGS = 100001
_EMBED_DIM = 96
_BATCH = 16384
_H0 = 50048
_H1A = 49920
_TAIL0 = _H0 + _H1A
_ICH = 2048


@functools.lru_cache(maxsize=None)
def _build_sc_gather():
    info = plsc.get_sparse_core_info()
    nc, ns = info.num_cores, info.num_subcores
    nw = nc * ns
    rows_per_w = _EMBED_DIM // nw
    n_ichunks = _BATCH // _ICH

    mesh = plsc.VectorSubcoreMesh(core_axis_name="c", subcore_axis_name="s")

    @functools.partial(
        pl.kernel,
        mesh=mesh,
        out_type=jax.ShapeDtypeStruct((_EMBED_DIM, _BATCH), jnp.float32),
        scratch_types=[
            pltpu.VMEM((1, _H0), jnp.float32),
            pltpu.VMEM((1, _H0), jnp.float32),
            pltpu.VMEM((_BATCH,), jnp.float32),
            pltpu.VMEM((2 * _ICH,), jnp.int32),
            pltpu.SemaphoreType.DMA,
            pltpu.SemaphoreType.DMA,
            pltpu.SemaphoreType.DMA,
            pltpu.SemaphoreType.DMA,
        ],
        compiler_params=pltpu.CompilerParams(needs_layout_passes=False),
    )
    def gather_kernel(idx_hbm, tab_t_hbm, tail_t_hbm, out_t_hbm, buf_a, buf_b,
                      outrow_v, idx_v, sem_a, sem_b, sem_i, sem_o):
        wid = lax.axis_index("s") * nc + lax.axis_index("c")
        zero_v = jnp.zeros((16,), jnp.int32)
        iota_v = lax.iota(jnp.int32, 16)

        def copy_a(c):
            return pltpu.make_async_copy(
                tab_t_hbm.at[pl.ds(c, 1), pl.ds(0, _H0)], buf_a, sem_a)

        def copy_b(c):
            return pltpu.make_async_copy(
                tab_t_hbm.at[pl.ds(c, 1), pl.ds(_H0, _H1A)],
                buf_b.at[:, pl.ds(0, _H1A)], sem_b)

        def copy_b_tail(c):
            return pltpu.make_async_copy(
                tail_t_hbm.at[pl.ds(c, 1), :],
                buf_b.at[:, pl.ds(_H1A, 128)], sem_b)

        def icopy(j):
            return pltpu.make_async_copy(
                idx_hbm.at[pl.ds(j * _ICH, _ICH)],
                idx_v.at[pl.ds((j % 2) * _ICH, _ICH)], sem_i)

        def ocopy(c):
            return pltpu.make_async_copy(outrow_v, out_t_hbm.at[c], sem_o)

        def start_b(c):
            copy_b(c).start()
            copy_b_tail(c).start()

        def wait_b(c):
            copy_b(c).wait()
            copy_b_tail(c).wait()

        c0 = wid * rows_per_w
        copy_a(c0).start()
        start_b(c0)

        def run_pass(body_vec):
            icopy(0).start()

            def chunk(j, carry):
                icopy(j).wait()

                @pl.when(j < n_ichunks - 1)
                def _():
                    icopy(j + 1).start()

                ib = (j % 2) * _ICH

                def group(i, carry2):
                    vecs = [
                        idx_v[pl.ds(ib + (i * 8 + g) * 16, 16)]
                        for g in range(8)
                    ]
                    pos0 = j * _ICH + i * 128
                    for g, vec in enumerate(vecs):
                        body_vec(vec, pos0 + g * 16)
                    return carry2

                lax.fori_loop(0, _ICH // 128, group, 0, unroll=1)
                return carry

            lax.fori_loop(0, n_ichunks, chunk, 0, unroll=1)

        for k in range(rows_per_w):
            c = c0 + k

            copy_a(c).wait()
            if k > 0:
                ocopy(c - 1).wait()

            def body_a(vec, pos):
                local = jnp.minimum(vec, _H0 - 1)
                g = plsc.load_gather(buf_a, [zero_v, local])
                outrow_v[pl.ds(pos, 16)] = g

            run_pass(body_a)
            if k + 1 < rows_per_w:
                copy_a(c + 1).start()

            wait_b(c)

            def body_b(vec, pos):
                local = vec - _H0
                m = local >= 0
                clamped = jnp.maximum(local, 0)
                g = plsc.load_gather(buf_b, [zero_v, clamped], mask=m)
                plsc.store_scatter(outrow_v, [iota_v + pos], g, mask=m)

            run_pass(body_b)
            if k + 1 < rows_per_w:
                start_b(c + 1)

            ocopy(c).start()

        ocopy(c0 + rows_per_w - 1).wait()

    return gather_kernel


@jax.jit
def kernel(viewer_indices, embedding_table):
    gather = _build_sc_gather()
    tail = jnp.pad(embedding_table[_TAIL0:], ((0, 128 - (_NUM_EMBEDDINGS - _TAIL0)), (0, 0)))
    out_t = gather(viewer_indices.astype(jnp.int32), embedding_table.T, tail.T)
    return out_t.T


# speedup vs baseline: 1.0705x; 1.0705x over previous
import functools

import jax
import jax.numpy as jnp
from jax import lax
from jax.experimental import pallas as pl
from jax.experimental.pallas import tpu as pltpu
from jax.experimental.pallas import tpu_sc as plsc

_NUM_EMBEDDINGS = 100001
_EMBED_DIM = 96
_BATCH = 16384
_H0 = 50048
_H1A = 49920
_TAIL0 = _H0 + _H1A
_ICH = 2048
_IRING = 4
_IAHEAD = 3
_SUBS_A = (12800, 12800, 12800, 11648)
_SUBS_B = (12800, 12800, 12800, 11520)


@functools.lru_cache(maxsize=None)
def _build_sc_gather():
    info = plsc.get_sparse_core_info()
    nc, ns = info.num_cores, info.num_subcores
    nw = nc * ns
    rows_per_w = _EMBED_DIM // nw
    n_ichunks = _BATCH // _ICH

    mesh = plsc.VectorSubcoreMesh(core_axis_name="c", subcore_axis_name="s")

    @functools.partial(
        pl.kernel,
        mesh=mesh,
        out_type=jax.ShapeDtypeStruct((_EMBED_DIM, _BATCH), jnp.float32),
        scratch_types=[
            pltpu.VMEM((1, _H0), jnp.float32),
            pltpu.VMEM((1, _H1A + 128), jnp.float32),
            pltpu.VMEM((_BATCH,), jnp.float32),
            pltpu.VMEM((_IRING * _ICH,), jnp.int32),
            pltpu.SemaphoreType.DMA,
            pltpu.SemaphoreType.DMA,
            pltpu.SemaphoreType.DMA,
            pltpu.SemaphoreType.DMA,
        ],
        compiler_params=pltpu.CompilerParams(needs_layout_passes=False),
    )
    def gather_kernel(idx_hbm, tab_t_hbm, tail_t_hbm, out_t_hbm, buf_a, buf_b,
                      outrow_v, idx_v, sem_a, sem_b, sem_i, sem_o):
        wid = lax.axis_index("s") * nc + lax.axis_index("c")
        zero_v = jnp.zeros((16,), jnp.int32)
        iota_v = lax.iota(jnp.int32, 16)

        def start_a(c):
            o = 0
            for n in _SUBS_A:
                pltpu.make_async_copy(
                    tab_t_hbm.at[pl.ds(c, 1), pl.ds(o, n)],
                    buf_a.at[:, pl.ds(o, n)], sem_a).start()
                o += n

        def wait_a():
            pltpu.make_async_copy(
                tab_t_hbm.at[pl.ds(0, 1), pl.ds(0, _H0)], buf_a, sem_a).wait()

        def start_b(c):
            o = 0
            for n in _SUBS_B:
                pltpu.make_async_copy(
                    tab_t_hbm.at[pl.ds(c, 1), pl.ds(_H0 + o, n)],
                    buf_b.at[:, pl.ds(o, n)], sem_b).start()
                o += n
            pltpu.make_async_copy(
                tail_t_hbm.at[pl.ds(c, 1), :],
                buf_b.at[:, pl.ds(_H1A, 128)], sem_b).start()

        def wait_b():
            pltpu.make_async_copy(
                tab_t_hbm.at[pl.ds(0, 1), pl.ds(0, _H1A + 128)],
                buf_b, sem_b).wait()

        def icopy(j, slot):
            return pltpu.make_async_copy(
                idx_hbm.at[pl.ds(j * _ICH, _ICH)],
                idx_v.at[pl.ds(slot * _ICH, _ICH)], sem_i)

        def prefire_idx():
            for j in range(_IAHEAD):
                icopy(j, j).start()

        def ocopy(c):
            return pltpu.make_async_copy(outrow_v, out_t_hbm.at[c], sem_o)

        c0 = wid * rows_per_w
        start_a(c0)
        prefire_idx()
        start_b(c0)

        def run_pass(body_vec):
            def chunk(j, carry):
                icopy(j, 0).wait()

                @pl.when(j < n_ichunks - _IAHEAD)
                def _():
                    icopy(j + _IAHEAD, (j + _IAHEAD) % _IRING).start()

                ib = pl.multiple_of((j % _IRING) * _ICH, _ICH)

                def group(i, carry2):
                    base = pl.multiple_of(ib + i * 128, 128)
                    vecs = [
                        idx_v[pl.ds(base + g * 16, 16)] for g in range(8)
                    ]
                    pos0 = pl.multiple_of(j * _ICH + i * 128, 128)
                    for g, vec in enumerate(vecs):
                        body_vec(vec, pos0 + g * 16)
                    return carry2

                lax.fori_loop(0, _ICH // 128, group, 0, unroll=1)
                return carry

            lax.fori_loop(0, n_ichunks, chunk, 0, unroll=1)

        for k in range(rows_per_w):
            c = c0 + k

            wait_a()
            if k > 0:
                ocopy(c - 1).wait()

            def body_a(vec, pos):
                local = jnp.minimum(vec, _H0 - 1)
                g = plsc.load_gather(buf_a, [zero_v, local])
                outrow_v[pl.ds(pos, 16)] = g

            run_pass(body_a)
            prefire_idx()
            if k + 1 < rows_per_w:
                start_a(c + 1)

            wait_b()

            def body_b(vec, pos):
                local = vec - _H0
                m = local >= 0
                clamped = jnp.maximum(local, 0)
                g = plsc.load_gather(buf_b, [zero_v, clamped], mask=m)
                plsc.store_scatter(outrow_v, [iota_v + pos], g, mask=m)

            run_pass(body_b)
            if k + 1 < rows_per_w:
                prefire_idx()
                start_b(c + 1)

            ocopy(c).start()

        ocopy(c0 + rows_per_w - 1).wait()

    return gather_kernel


@jax.jit
def kernel(viewer_indices, embedding_table):
    gather = _build_sc_gather()
    tail = jnp.pad(
        embedding_table[_TAIL0:],
        ((0, 128 - (_NUM_EMBEDDINGS - _TAIL0)), (0, 0)),
    )
    out_t = gather(viewer_indices.astype(jnp.int32), embedding_table.T,
                   tail.T)
    return out_t.T


# speedup vs baseline: 1.5831x; 1.4789x over previous
import functools

import jax
import jax.numpy as jnp
from jax import lax
from jax.experimental import pallas as pl
from jax.experimental.pallas import tpu as pltpu
from jax.experimental.pallas import tpu_sc as plsc

_NUM_EMBEDDINGS = 100001
_EMBED_DIM = 96
_BATCH = 16384
_CHUNK = 4096


@functools.lru_cache(maxsize=None)
def _build_sc_gather():
    info = plsc.get_sparse_core_info()
    nc, ns = info.num_cores, info.num_subcores
    nw = nc * ns
    rows_per_w = _EMBED_DIM // nw
    assert _EMBED_DIM % nw == 0 and _BATCH % _CHUNK == 0

    mesh = plsc.VectorSubcoreMesh(core_axis_name="c", subcore_axis_name="s")

    @functools.partial(
        pl.kernel,
        mesh=mesh,
        out_type=jax.ShapeDtypeStruct((_EMBED_DIM, _BATCH), jnp.float32),
        scratch_types=[
            pltpu.VMEM((_BATCH,), jnp.int32),
            pltpu.VMEM((1, _NUM_EMBEDDINGS), jnp.float32),
            pltpu.VMEM((2, _CHUNK), jnp.float32),
            pltpu.SemaphoreType.DMA,
            pltpu.SemaphoreType.DMA,
        ],
        compiler_params=pltpu.CompilerParams(needs_layout_passes=False),
    )
    def gather_kernel(idx_hbm, tab_t_hbm, out_hbm, idx_v, row_v, stage_v,
                      row_sem, out_sem):
        wid = lax.axis_index("s") * nc + lax.axis_index("c")
        pltpu.sync_copy(idx_hbm, idx_v)
        zero_v = jnp.zeros((16,), jnp.int32)

        for k in range(rows_per_w):
            c = wid * rows_per_w + k
            pltpu.async_copy(
                tab_t_hbm.at[pl.ds(c, 1), :], row_v, row_sem
            ).wait()

            for h in range(_BATCH // _CHUNK):
                buf = h % 2
                if h >= 2:
                    pltpu.make_async_copy(
                        stage_v.at[buf], out_hbm.at[c, pl.ds(0, _CHUNK)],
                        out_sem,
                    ).wait()

                def gather_group(i, carry, h=h, buf=buf):
                    vecs = [
                        idx_v[pl.ds(h * _CHUNK + (i * 8 + j) * 16, 16)]
                        for j in range(8)
                    ]
                    gs = [plsc.load_gather(row_v, [zero_v, v]) for v in vecs]
                    for j, g in enumerate(gs):
                        stage_v[buf, pl.ds((i * 8 + j) * 16, 16)] = g
                    return carry

                lax.fori_loop(0, _CHUNK // 128, gather_group, 0, unroll=1)
                pltpu.make_async_copy(
                    stage_v.at[buf],
                    out_hbm.at[c, pl.ds(h * _CHUNK, _CHUNK)],
                    out_sem,
                ).start()

            for _ in range(2):
                pltpu.make_async_copy(
                    stage_v.at[0], out_hbm.at[c, pl.ds(0, _CHUNK)], out_sem
                ).wait()

    return gather_kernel


@jax.jit
def kernel(viewer_indices, embedding_table):
    gather = _build_sc_gather()
    out_t = gather(viewer_indices.astype(jnp.int32), embedding_table.T)
    return out_t.T


# speedup vs baseline: 1.6013x; 1.0115x over previous
import functools

import jax
import jax.numpy as jnp
from jax import lax
from jax.experimental import pallas as pl
from jax.experimental.pallas import tpu as pltpu
from jax.experimental.pallas import tpu_sc as plsc

_NUM_EMBEDDINGS = 100001
_EMBED_DIM = 96
_BATCH = 16384
_CHUNK = 4096
_TAIL0 = 99968


@functools.lru_cache(maxsize=None)
def _build_sc_gather():
    info = plsc.get_sparse_core_info()
    nc, ns = info.num_cores, info.num_subcores
    nw = nc * ns
    rows_per_w = _EMBED_DIM // nw
    assert _EMBED_DIM % nw == 0 and _BATCH % _CHUNK == 0

    mesh = plsc.VectorSubcoreMesh(core_axis_name="c", subcore_axis_name="s")

    @functools.partial(
        pl.kernel,
        mesh=mesh,
        out_type=jax.ShapeDtypeStruct((_EMBED_DIM, _BATCH), jnp.float32),
        scratch_types=[
            pltpu.VMEM((_BATCH,), jnp.int32),
            pltpu.VMEM((1, 100096), jnp.float32),
            pltpu.VMEM((2, _CHUNK), jnp.float32),
            pltpu.SemaphoreType.DMA,
            pltpu.SemaphoreType.DMA,
            pltpu.SemaphoreType.DMA,
        ],
        compiler_params=pltpu.CompilerParams(needs_layout_passes=False),
    )
    def gather_kernel(idx_hbm, tab_t_hbm, tail_t_hbm, out_hbm, idx_v, row_v,
                      stage_v, row_sem, idx_sem, out_sem):
        wid = lax.axis_index("s") * nc + lax.axis_index("c")
        zero_v = jnp.zeros((16,), jnp.int32)
        subs = ((0, 25088), (25088, 25088), (50176, 25088), (75264, 24704))

        def start_row(c):
            for o, n in subs:
                pltpu.make_async_copy(
                    tab_t_hbm.at[pl.ds(c, 1), pl.ds(o, n)],
                    row_v.at[:, pl.ds(o, n)], row_sem).start()
            pltpu.make_async_copy(
                tail_t_hbm.at[pl.ds(c, 1), :],
                row_v.at[:, pl.ds(_TAIL0, 128)], row_sem).start()

        def wait_row():
            pltpu.make_async_copy(
                tab_t_hbm.at[pl.ds(0, 1), pl.ds(0, _TAIL0 + 128)],
                row_v, row_sem).wait()

        c0 = wid * rows_per_w
        start_row(c0)
        pltpu.async_copy(idx_hbm, idx_v, idx_sem).wait()

        for k in range(rows_per_w):
            c = c0 + k
            wait_row()

            for h in range(_BATCH // _CHUNK):
                buf = h % 2
                if h >= 2:
                    pltpu.make_async_copy(
                        stage_v.at[buf], out_hbm.at[c, pl.ds(0, _CHUNK)],
                        out_sem,
                    ).wait()

                def gather_group(i, carry, h=h, buf=buf):
                    vecs = [
                        idx_v[pl.ds(h * _CHUNK + (i * 8 + j) * 16, 16)]
                        for j in range(8)
                    ]
                    gs = [plsc.load_gather(row_v, [zero_v, v]) for v in vecs]
                    for j, g in enumerate(gs):
                        stage_v[buf, pl.ds((i * 8 + j) * 16, 16)] = g
                    return carry

                lax.fori_loop(0, _CHUNK // 128, gather_group, 0, unroll=1)
                pltpu.make_async_copy(
                    stage_v.at[buf],
                    out_hbm.at[c, pl.ds(h * _CHUNK, _CHUNK)],
                    out_sem,
                ).start()

            if k + 1 < rows_per_w:
                start_row(c + 1)

            for _ in range(2):
                pltpu.make_async_copy(
                    stage_v.at[0], out_hbm.at[c, pl.ds(0, _CHUNK)], out_sem
                ).wait()

    return gather_kernel


@jax.jit
def kernel(viewer_indices, embedding_table):
    gather = _build_sc_gather()
    tail = jnp.pad(
        embedding_table[_TAIL0:],
        ((0, 128 - (_NUM_EMBEDDINGS - _TAIL0)), (0, 0)),
    )
    out_t = gather(viewer_indices.astype(jnp.int32), embedding_table.T,
                   tail.T)
    return out_t.T
